# Initial kernel scaffold; baseline (speedup 1.0000x reference)
import jax
import jax.numpy as jnp
from jax.experimental import pallas as pl


def kernel(n, segment_ids, W1, b1, gamma, beta, W2, b2):
    raise NotImplementedError("write your pallas kernel here")


# speedup vs baseline: 7.4341x; 7.4341x over previous
import jax
import jax.numpy as jnp
from jax.experimental import pallas as pl
from jax.experimental.pallas import tpu as pltpu

N_ROWS = 320000
D = 128
NSEG = 512
BLK = 2560
NBLK = N_ROWS // BLK
EPS = 1e-5


def _body(seg_ref, n_ref, W1_ref, b1_ref, gamma_ref, beta_ref, W2_ref,
          b2_ref, y_ref, acc_ref):
    i = pl.program_id(0)

    @pl.when(i == 0)
    def _init():
        acc_ref[...] = jnp.zeros_like(acc_ref)

    x = n_ref[...]
    h = jnp.dot(x, W1_ref[...], preferred_element_type=jnp.float32)
    h = h + b1_ref[...]
    a = h[:, :D]
    g = h[:, D:]
    out = a * jax.nn.sigmoid(g)

    ids = seg_ref[0]
    onehot = (jax.lax.broadcasted_iota(jnp.int32, (NSEG, BLK), 0)
              == ids).astype(jnp.float32)
    acc_ref[...] += jnp.dot(onehot, out, preferred_element_type=jnp.float32)

    @pl.when(i == NBLK - 1)
    def _finish():
        r = acc_ref[...]
        mean = jnp.mean(r, axis=0, keepdims=True)
        var = jnp.mean((r - mean) ** 2, axis=0, keepdims=True)
        bn = (r - mean) * jax.lax.rsqrt(var + EPS) * gamma_ref[...] + beta_ref[...]
        y_ref[...] = (jnp.dot(bn, W2_ref[...], preferred_element_type=jnp.float32)
                      + b2_ref[...])


def kernel(n, segment_ids, W1, b1, gamma, beta, W2, b2):
    seg = segment_ids.astype(jnp.int32).reshape(NBLK, 1, BLK)
    b1r = b1.reshape(1, 2 * D)
    gr = gamma.reshape(1, D)
    br = beta.reshape(1, D)
    b2r = b2.reshape(1, D)
    y = pl.pallas_call(
        _body,
        grid=(NBLK,),
        in_specs=[
            pl.BlockSpec((1, 1, BLK), lambda i: (i, 0, 0)),
            pl.BlockSpec((BLK, D), lambda i: (i, 0)),
            pl.BlockSpec((D, 2 * D), lambda i: (0, 0)),
            pl.BlockSpec((1, 2 * D), lambda i: (0, 0)),
            pl.BlockSpec((1, D), lambda i: (0, 0)),
            pl.BlockSpec((1, D), lambda i: (0, 0)),
            pl.BlockSpec((D, D), lambda i: (0, 0)),
            pl.BlockSpec((1, D), lambda i: (0, 0)),
        ],
        out_specs=pl.BlockSpec((NSEG, D), lambda i: (0, 0)),
        out_shape=jax.ShapeDtypeStruct((NSEG, D), jnp.float32),
        scratch_shapes=[pltpu.VMEM((NSEG, D), jnp.float32)],
    )(seg, n, W1, b1r, gr, br, W2, b2r)
    return y
